# baseline (speedup 1.0000x reference)
import dataclasses
import functools

import jax
import jax.numpy as jnp
from jax import lax
from jax.experimental import pallas as pl
from jax.experimental.pallas import tpu as pltpu
from jax.experimental.pallas import tpu_sc as plsc

N = 10000
E = 160000
B = 64
ATOM = 118
D = 256
OUT = 100

NP_ = 10240
KP = 128
ROWS = 1024
NBLK = NP_ // ROWS

NC = 2
NS = 16
L = 16
DH = D // NC
EBLK = 128
BLKS = 80
E_PAD = NS * BLKS * EBLK
EPS_ = E_PAD // NS
NROWS_S = NP_ // NS


def _bcast_lane(vec16, e):
    idx = jnp.full((L, 1), e, jnp.int32)
    dn = lax.GatherDimensionNumbers(
        offset_dims=(), collapsed_slice_dims=(0,), start_index_map=(0,))
    return lax.gather(vec16, idx, dn, (1,),
                      mode=lax.GatherScatterMode.PROMISE_IN_BOUNDS)


def _edge_body(hb_lo, hb_hi, pk_hbm, lx_hbm, w_lo, w_hi, z_hbm,
               out_lo, out_hi, pk0, pk1, lx0, lx1, rows0, rows1, db0, db1,
               w_v, isem0, isem1, gsem0, gsem1, ssem0, ssem1, acc_sh):
    c = lax.axis_index("c")
    s = lax.axis_index("s")
    pk = (pk0, pk1)
    lx = (lx0, lx1)
    db = (db0, db1)
    ssem = (ssem0, ssem1)
    rows = (rows0, rows1)
    isem = (isem0, isem1)
    gsem = (gsem0, gsem1)

    pltpu.sync_copy(z_hbm, acc_sh.at[pl.ds(s * NROWS_S, NROWS_S)])

    @pl.when(c == 0)
    def _():
        pltpu.sync_copy(w_lo, w_v)

    @pl.when(c == 1)
    def _():
        pltpu.sync_copy(w_hi, w_v)

    plsc.subcore_barrier()

    def run(h_ref):
        w_regs = [w_v[pl.ds(16 * k, 16)] for k in range(DH // L)]

        def idx_start(blk, b):
            base = (s * BLKS + blk) * EBLK
            pltpu.make_async_copy(
                pk_hbm.at[s * BLKS + blk], pk[b], isem[b]).start()
            pltpu.make_async_copy(
                lx_hbm.at[pl.ds(base * L, EBLK * L)], lx[b], isem[b]).start()

        def idx_wait(b):
            pltpu.make_async_copy(pk_hbm.at[0], pk[b], isem[b]).wait()
            pltpu.make_async_copy(
                lx_hbm.at[pl.ds(0, EBLK * L)], lx[b], isem[b]).wait()

        def gather_start(b):
            pltpu.make_async_copy(
                h_ref.at[pk[b].at[0]], rows[b], gsem[b]).start()

        def gather_wait(b):
            pltpu.make_async_copy(
                h_ref.at[pk[b].at[0]], rows[b], gsem[b]).wait()

        def scatter_start(b):
            pltpu.sync_copy(rows[b], acc_sh.at[pk[b].at[1]], add=True)

        def scatter_wait(b):
            pass

        def compute(b):
            rows_v = rows[b]
            lx_v = lx[b]

            @pl.loop(0, EBLK // L)
            def _(i16):
                for e in range(L):
                    r = i16 * L + e
                    bc = lx_v[pl.ds(r * L, L)]
                    for seg in range(DH // L):
                        sl = rows_v[r, pl.ds(seg * L, L)]
                        rows_v[r, pl.ds(seg * L, L)] = jnp.maximum(
                            sl + bc * w_regs[seg], 0.0)

        idx_start(0, 0)
        idx_start(1, 1)
        idx_wait(0)
        gather_start(0)

        @pl.loop(0, BLKS, step=2)
        def _(blk):
            for b in range(2):
                i = blk + b
                bn = 1 - b
                gather_wait(b)

                @pl.when(i + 1 < BLKS)
                def _():
                    idx_wait(bn)

                    @pl.when(i >= 1)
                    def _():
                        scatter_wait(bn)

                    gather_start(bn)

                compute(b)
                scatter_start(b)

                @pl.when(i + 2 < BLKS)
                def _():
                    idx_start(i + 2, b)

        scatter_wait(0)
        scatter_wait(1)

    @pl.when(c == 0)
    def _():
        run(hb_lo)

    @pl.when(c == 1)
    def _():
        run(hb_hi)

    plsc.subcore_barrier()

    sl = pl.ds(s * NROWS_S, NROWS_S)

    @pl.when(c == 0)
    def _():
        pltpu.sync_copy(acc_sh.at[sl], out_lo.at[sl])

    @pl.when(c == 1)
    def _():
        pltpu.sync_copy(acc_sh.at[sl], out_hi.at[sl])


def _t1_body(a_ref, w_ref, b_ref, lo_ref, hi_ref):
    h = jnp.dot(a_ref[...], w_ref[...], preferred_element_type=jnp.float32)
    h = h + b_ref[...]
    lo_ref[...] = h[:, :128]
    hi_ref[...] = h[:, 128:]


def _t2_body(lo_ref, hi_ref, alo_ref, ahi_ref, gid_ref, wg_ref, bg_ref,
             be_ref, eps_ref, feat_ref):
    i = pl.program_id(0)

    @pl.when(i == 0)
    def _():
        feat_ref[...] = jnp.zeros_like(feat_ref)

    h = jnp.concatenate([lo_ref[...], hi_ref[...]], axis=1) - be_ref[...]
    a = jnp.concatenate([alo_ref[...], ahi_ref[...]], axis=1)
    x = (1.0 + eps_ref[0, 0]) * h + a
    t = jnp.dot(x, wg_ref[...], preferred_element_type=jnp.float32)
    gids = gid_ref[0, 0, :]
    rows = jax.lax.broadcasted_iota(jnp.int32, (B, ROWS), 0)
    mask = (gids[None, :] == rows).astype(jnp.float32)
    counts = jnp.sum(mask, axis=1)
    acc = jnp.dot(mask, t, preferred_element_type=jnp.float32)
    feat_ref[...] += acc + counts[:, None] * bg_ref[...]


def _t3_body(feat_ref, epsu_ref, wm_ref, bm_ref, out_ref):
    feat = feat_ref[...]
    mx = jnp.max(feat)
    mn = jnp.min(feat)
    f = (feat - mn) / (epsu_ref[0, 0] + mx - mn)
    out_ref[...] = jnp.dot(f, wm_ref[...],
                           preferred_element_type=jnp.float32) + bm_ref[...]


def kernel(atomic_num, edge_index, edge_length, graph_ids, W_node, b_node,
           W_edge, b_edge, W_gin, b_gin, eps_gin, eps_u, W_mlp, b_mlp):
    f32 = jnp.float32
    a_pad = jnp.zeros((NP_, KP), f32).at[:N, :ATOM].set(atomic_num)
    w_pad = jnp.zeros((KP, D), f32).at[:ATOM, :].set(W_node)

    hb_lo, hb_hi = pl.pallas_call(
        _t1_body,
        grid=(NBLK,),
        in_specs=[
            pl.BlockSpec((ROWS, KP), lambda i: (i, 0)),
            pl.BlockSpec((KP, D), lambda i: (0, 0)),
            pl.BlockSpec((1, D), lambda i: (0, 0)),
        ],
        out_specs=[
            pl.BlockSpec((ROWS, 128), lambda i: (i, 0)),
            pl.BlockSpec((ROWS, 128), lambda i: (i, 0)),
        ],
        out_shape=[
            jax.ShapeDtypeStruct((NP_, 128), f32),
            jax.ShapeDtypeStruct((NP_, 128), f32),
        ],
    )(a_pad, w_pad, (b_node + b_edge).reshape(1, D))

    src_pad = jnp.zeros((E_PAD,), jnp.int32).at[:E].set(edge_index[0])
    dst_pad = jnp.full((E_PAD,), N, jnp.int32).at[:E].set(edge_index[1])
    len_pad = jnp.zeros((E_PAD,), f32).at[:E].set(edge_length)
    packed = jnp.stack([
        src_pad.reshape(-1, EBLK),
        dst_pad.reshape(-1, EBLK),
    ], axis=1)
    lenexp = jnp.broadcast_to(len_pad[:, None], (E_PAD, L)).reshape(-1)
    zeros = jnp.zeros((NROWS_S, DH), f32)
    mesh = plsc.VectorSubcoreMesh(core_axis_name="c", subcore_axis_name="s")

    cp = pltpu.CompilerParams()
    if "needs_layout_passes" in pltpu.CompilerParams.__dataclass_fields__:
        cp = dataclasses.replace(cp, needs_layout_passes=False)

    edge_call = pl.kernel(
        _edge_body,
        compiler_params=cp,
        out_type=[
            jax.ShapeDtypeStruct((NP_, DH), f32),
            jax.ShapeDtypeStruct((NP_, DH), f32),
        ],
        mesh=mesh,
        scratch_types=[
            pltpu.VMEM((2, EBLK), jnp.int32),
            pltpu.VMEM((2, EBLK), jnp.int32),
            pltpu.VMEM((EBLK * L,), f32),
            pltpu.VMEM((EBLK * L,), f32),
            pltpu.VMEM((EBLK, DH), f32),
            pltpu.VMEM((EBLK, DH), f32),
            pltpu.VMEM((1, EBLK), jnp.int32),
            pltpu.VMEM((1, EBLK), jnp.int32),
            pltpu.VMEM((DH,), f32),
            pltpu.SemaphoreType.DMA,
            pltpu.SemaphoreType.DMA,
            pltpu.SemaphoreType.DMA,
            pltpu.SemaphoreType.DMA,
            pltpu.SemaphoreType.DMA,
            pltpu.SemaphoreType.DMA,
            pltpu.VMEM_SHARED((NP_, DH), f32),
        ],
    )
    agg_lo, agg_hi = edge_call(hb_lo, hb_hi, packed, lenexp,
                               W_edge[0, :DH], W_edge[0, DH:], zeros)

    gids_pad = jnp.full((NP_,), B, jnp.int32).at[:N].set(graph_ids)
    gids3 = gids_pad.reshape(NBLK, 1, ROWS)

    feat = pl.pallas_call(
        _t2_body,
        grid=(NBLK,),
        in_specs=[
            pl.BlockSpec((ROWS, 128), lambda i: (i, 0)),
            pl.BlockSpec((ROWS, 128), lambda i: (i, 0)),
            pl.BlockSpec((ROWS, 128), lambda i: (i, 0)),
            pl.BlockSpec((ROWS, 128), lambda i: (i, 0)),
            pl.BlockSpec((1, 1, ROWS), lambda i: (i, 0, 0)),
            pl.BlockSpec((D, D), lambda i: (0, 0)),
            pl.BlockSpec((1, D), lambda i: (0, 0)),
            pl.BlockSpec((1, D), lambda i: (0, 0)),
            pl.BlockSpec((1, 1), lambda i: (0, 0)),
        ],
        out_specs=pl.BlockSpec((B, D), lambda i: (0, 0)),
        out_shape=jax.ShapeDtypeStruct((B, D), f32),
    )(hb_lo, hb_hi, agg_lo, agg_hi, gids3, W_gin, b_gin.reshape(1, D),
      b_edge.reshape(1, D), eps_gin.reshape(1, 1))

    wm_pad = jnp.zeros((D, 128), f32).at[:, :OUT].set(W_mlp)
    bm_pad = jnp.zeros((1, 128), f32).at[0, :OUT].set(b_mlp)

    out_pad = pl.pallas_call(
        _t3_body,
        in_specs=[
            pl.BlockSpec((B, D), lambda: (0, 0)),
            pl.BlockSpec((1, 1), lambda: (0, 0)),
            pl.BlockSpec((D, 128), lambda: (0, 0)),
            pl.BlockSpec((1, 128), lambda: (0, 0)),
        ],
        out_specs=pl.BlockSpec((B, 128), lambda: (0, 0)),
        out_shape=jax.ShapeDtypeStruct((B, 128), f32),
    )(feat, eps_u.reshape(1, 1), wm_pad, bm_pad)

    return out_pad[:, :OUT]


# speedup vs baseline: 1.5771x; 1.5771x over previous
import dataclasses
import functools

import jax
import jax.numpy as jnp
from jax import lax
from jax.experimental import pallas as pl
from jax.experimental.pallas import tpu as pltpu
from jax.experimental.pallas import tpu_sc as plsc

N = 10000
E = 160000
B = 64
ATOM = 118
D = 256
OUT = 100

NP_ = 10240
KP = 128
ROWS = 1024
NBLK = NP_ // ROWS

NC = 2
NS = 16
L = 16
DH = D // NC
EBLK = 128
BLKS = 80
E_PAD = NS * BLKS * EBLK
EPS_ = E_PAD // NS
NROWS_S = NP_ // NS


def _bcast_lane(vec16, e):
    idx = jnp.full((L, 1), e, jnp.int32)
    dn = lax.GatherDimensionNumbers(
        offset_dims=(), collapsed_slice_dims=(0,), start_index_map=(0,))
    return lax.gather(vec16, idx, dn, (1,),
                      mode=lax.GatherScatterMode.PROMISE_IN_BOUNDS)


def _edge_body(hb_lo, hb_hi, pk_hbm, w_lo, w_hi, z_hbm,
               out_lo, out_hi, pk0, pk1, rows0, rows1, db0, db1,
               w_v, isem0, isem1, gsem0, gsem1, ssem0, ssem1, acc_sh):
    c = lax.axis_index("c")
    s = lax.axis_index("s")
    pk = (pk0, pk1)
    db = (db0, db1)
    ssem = (ssem0, ssem1)
    rows = (rows0, rows1)
    isem = (isem0, isem1)
    gsem = (gsem0, gsem1)

    pltpu.sync_copy(z_hbm, acc_sh.at[pl.ds(s * NROWS_S, NROWS_S)])

    @pl.when(c == 0)
    def _():
        pltpu.sync_copy(w_lo, w_v)

    @pl.when(c == 1)
    def _():
        pltpu.sync_copy(w_hi, w_v)

    plsc.subcore_barrier()

    def run(h_ref):
        w_regs = [w_v[pl.ds(16 * k, 16)] for k in range(DH // L)]

        def idx_start(blk, b):
            pltpu.make_async_copy(
                pk_hbm.at[s * BLKS + blk], pk[b], isem[b]).start()

        def idx_wait(b):
            pltpu.make_async_copy(pk_hbm.at[0], pk[b], isem[b]).wait()

        def gather_start(b):
            pltpu.make_async_copy(
                h_ref.at[pk[b].at[0]], rows[b], gsem[b]).start()

        def gather_wait(b):
            pltpu.make_async_copy(
                h_ref.at[pk[b].at[0]], rows[b], gsem[b]).wait()

        def scatter_start(b):
            for k in range(EBLK // L):
                db[b][0, pl.ds(k * L, L)] = pk[b][1, pl.ds(k * L, L)]
            pltpu.make_async_copy(
                rows[b], acc_sh.at[db[b].at[0]], ssem[b]).start(add=True)

        def scatter_wait(b):
            pltpu.make_async_copy(
                rows[b], acc_sh.at[db[b].at[0]], ssem[b]).wait()

        def compute(b):
            rows_v = rows[b]
            pk_v = pk[b]

            @pl.loop(0, EBLK // L)
            def _(i16):
                len16 = plsc.bitcast(pk_v[2, pl.ds(i16 * L, L)], jnp.float32)
                for e in range(L):
                    bc = _bcast_lane(len16, e)
                    r = i16 * L + e
                    for seg in range(DH // L):
                        sl = rows_v[r, pl.ds(seg * L, L)]
                        rows_v[r, pl.ds(seg * L, L)] = jnp.maximum(
                            sl + bc * w_regs[seg], 0.0)

        idx_start(0, 0)
        idx_start(1, 1)
        idx_wait(0)
        gather_start(0)

        @pl.loop(0, BLKS, step=2)
        def _(blk):
            for b in range(2):
                i = blk + b
                bn = 1 - b
                gather_wait(b)

                @pl.when(i + 1 < BLKS)
                def _():
                    idx_wait(bn)

                    @pl.when(i >= 1)
                    def _():
                        scatter_wait(bn)

                    gather_start(bn)

                compute(b)
                scatter_start(b)

                @pl.when(i + 2 < BLKS)
                def _():
                    idx_start(i + 2, b)

        scatter_wait(0)
        scatter_wait(1)

    @pl.when(c == 0)
    def _():
        run(hb_lo)

    @pl.when(c == 1)
    def _():
        run(hb_hi)

    plsc.subcore_barrier()

    sl = pl.ds(s * NROWS_S, NROWS_S)

    @pl.when(c == 0)
    def _():
        pltpu.sync_copy(acc_sh.at[sl], out_lo.at[sl])

    @pl.when(c == 1)
    def _():
        pltpu.sync_copy(acc_sh.at[sl], out_hi.at[sl])


def _t1_body(a_ref, w_ref, b_ref, lo_ref, hi_ref):
    h = jnp.dot(a_ref[...], w_ref[...], preferred_element_type=jnp.float32)
    h = h + b_ref[...]
    lo_ref[...] = h[:, :128]
    hi_ref[...] = h[:, 128:]


def _t2_body(lo_ref, hi_ref, alo_ref, ahi_ref, gid_ref, wg_ref, bg_ref,
             be_ref, eps_ref, feat_ref):
    i = pl.program_id(0)

    @pl.when(i == 0)
    def _():
        feat_ref[...] = jnp.zeros_like(feat_ref)

    h = jnp.concatenate([lo_ref[...], hi_ref[...]], axis=1) - be_ref[...]
    a = jnp.concatenate([alo_ref[...], ahi_ref[...]], axis=1)
    x = (1.0 + eps_ref[0, 0]) * h + a
    t = jnp.dot(x, wg_ref[...], preferred_element_type=jnp.float32)
    gids = gid_ref[0, 0, :]
    rows = jax.lax.broadcasted_iota(jnp.int32, (B, ROWS), 0)
    mask = (gids[None, :] == rows).astype(jnp.float32)
    counts = jnp.sum(mask, axis=1)
    acc = jnp.dot(mask, t, preferred_element_type=jnp.float32)
    feat_ref[...] += acc + counts[:, None] * bg_ref[...]


def _t3_body(feat_ref, epsu_ref, wm_ref, bm_ref, out_ref):
    feat = feat_ref[...]
    mx = jnp.max(feat)
    mn = jnp.min(feat)
    f = (feat - mn) / (epsu_ref[0, 0] + mx - mn)
    out_ref[...] = jnp.dot(f, wm_ref[...],
                           preferred_element_type=jnp.float32) + bm_ref[...]


def kernel(atomic_num, edge_index, edge_length, graph_ids, W_node, b_node,
           W_edge, b_edge, W_gin, b_gin, eps_gin, eps_u, W_mlp, b_mlp):
    f32 = jnp.float32
    a_pad = jnp.zeros((NP_, KP), f32).at[:N, :ATOM].set(atomic_num)
    w_pad = jnp.zeros((KP, D), f32).at[:ATOM, :].set(W_node)

    hb_lo, hb_hi = pl.pallas_call(
        _t1_body,
        grid=(NBLK,),
        in_specs=[
            pl.BlockSpec((ROWS, KP), lambda i: (i, 0)),
            pl.BlockSpec((KP, D), lambda i: (0, 0)),
            pl.BlockSpec((1, D), lambda i: (0, 0)),
        ],
        out_specs=[
            pl.BlockSpec((ROWS, 128), lambda i: (i, 0)),
            pl.BlockSpec((ROWS, 128), lambda i: (i, 0)),
        ],
        out_shape=[
            jax.ShapeDtypeStruct((NP_, 128), f32),
            jax.ShapeDtypeStruct((NP_, 128), f32),
        ],
    )(a_pad, w_pad, (b_node + b_edge).reshape(1, D))

    src_pad = jnp.zeros((E_PAD,), jnp.int32).at[:E].set(edge_index[0])
    dst_pad = jnp.full((E_PAD,), N, jnp.int32).at[:E].set(edge_index[1])
    len_pad = jnp.zeros((E_PAD,), f32).at[:E].set(edge_length)
    packed = jnp.stack([
        src_pad.reshape(-1, EBLK),
        dst_pad.reshape(-1, EBLK),
        jax.lax.bitcast_convert_type(len_pad, jnp.int32).reshape(-1, EBLK),
    ], axis=1)
    zeros = jnp.zeros((NROWS_S, DH), f32)
    mesh = plsc.VectorSubcoreMesh(core_axis_name="c", subcore_axis_name="s")

    cp = pltpu.CompilerParams()
    if "needs_layout_passes" in pltpu.CompilerParams.__dataclass_fields__:
        cp = dataclasses.replace(cp, needs_layout_passes=False)

    edge_call = pl.kernel(
        _edge_body,
        compiler_params=cp,
        out_type=[
            jax.ShapeDtypeStruct((NP_, DH), f32),
            jax.ShapeDtypeStruct((NP_, DH), f32),
        ],
        mesh=mesh,
        scratch_types=[
            pltpu.VMEM((3, EBLK), jnp.int32),
            pltpu.VMEM((3, EBLK), jnp.int32),
            pltpu.VMEM((EBLK, DH), f32),
            pltpu.VMEM((EBLK, DH), f32),
            pltpu.VMEM((1, EBLK), jnp.int32),
            pltpu.VMEM((1, EBLK), jnp.int32),
            pltpu.VMEM((DH,), f32),
            pltpu.SemaphoreType.DMA,
            pltpu.SemaphoreType.DMA,
            pltpu.SemaphoreType.DMA,
            pltpu.SemaphoreType.DMA,
            pltpu.SemaphoreType.DMA,
            pltpu.SemaphoreType.DMA,
            pltpu.VMEM_SHARED((NP_, DH), f32),
        ],
    )
    agg_lo, agg_hi = edge_call(hb_lo, hb_hi, packed,
                               W_edge[0, :DH], W_edge[0, DH:], zeros)

    gids_pad = jnp.full((NP_,), B, jnp.int32).at[:N].set(graph_ids)
    gids3 = gids_pad.reshape(NBLK, 1, ROWS)

    feat = pl.pallas_call(
        _t2_body,
        grid=(NBLK,),
        in_specs=[
            pl.BlockSpec((ROWS, 128), lambda i: (i, 0)),
            pl.BlockSpec((ROWS, 128), lambda i: (i, 0)),
            pl.BlockSpec((ROWS, 128), lambda i: (i, 0)),
            pl.BlockSpec((ROWS, 128), lambda i: (i, 0)),
            pl.BlockSpec((1, 1, ROWS), lambda i: (i, 0, 0)),
            pl.BlockSpec((D, D), lambda i: (0, 0)),
            pl.BlockSpec((1, D), lambda i: (0, 0)),
            pl.BlockSpec((1, D), lambda i: (0, 0)),
            pl.BlockSpec((1, 1), lambda i: (0, 0)),
        ],
        out_specs=pl.BlockSpec((B, D), lambda i: (0, 0)),
        out_shape=jax.ShapeDtypeStruct((B, D), f32),
    )(hb_lo, hb_hi, agg_lo, agg_hi, gids3, W_gin, b_gin.reshape(1, D),
      b_edge.reshape(1, D), eps_gin.reshape(1, 1))

    wm_pad = jnp.zeros((D, 128), f32).at[:, :OUT].set(W_mlp)
    bm_pad = jnp.zeros((1, 128), f32).at[0, :OUT].set(b_mlp)

    out_pad = pl.pallas_call(
        _t3_body,
        in_specs=[
            pl.BlockSpec((B, D), lambda: (0, 0)),
            pl.BlockSpec((1, 1), lambda: (0, 0)),
            pl.BlockSpec((D, 128), lambda: (0, 0)),
            pl.BlockSpec((1, 128), lambda: (0, 0)),
        ],
        out_specs=pl.BlockSpec((B, 128), lambda: (0, 0)),
        out_shape=jax.ShapeDtypeStruct((B, 128), f32),
    )(feat, eps_u.reshape(1, 1), wm_pad, bm_pad)

    return out_pad[:, :OUT]


# speedup vs baseline: 1.5840x; 1.0044x over previous
import dataclasses
import functools

import jax
import jax.numpy as jnp
from jax import lax
from jax.experimental import pallas as pl
from jax.experimental.pallas import tpu as pltpu
from jax.experimental.pallas import tpu_sc as plsc

N = 10000
E = 160000
B = 64
ATOM = 118
D = 256
OUT = 100

NP_ = 10240
KP = 128
ROWS = 1024
NBLK = NP_ // ROWS

NC = 2
NS = 16
L = 16
DH = D // NC
EBLK = 128
BLKS = 80
E_PAD = NS * BLKS * EBLK
EPS_ = E_PAD // NS
NROWS_S = NP_ // NS


def _bcast_lane(vec16, e):
    idx = jnp.full((L, 1), e, jnp.int32)
    dn = lax.GatherDimensionNumbers(
        offset_dims=(), collapsed_slice_dims=(0,), start_index_map=(0,))
    return lax.gather(vec16, idx, dn, (1,),
                      mode=lax.GatherScatterMode.PROMISE_IN_BOUNDS)


def _edge_body(hb_lo, hb_hi, pk_hbm, w_lo, w_hi, z_hbm,
               out_lo, out_hi, pk0, pk1, rows0, rows1, db0, db1,
               w_v, isem0, isem1, gsem0, gsem1, ssem0, ssem1, acc_sh):
    c = lax.axis_index("c")
    s = lax.axis_index("s")
    pk = (pk0, pk1)
    db = (db0, db1)
    ssem = (ssem0, ssem1)
    rows = (rows0, rows1)
    isem = (isem0, isem1)
    gsem = (gsem0, gsem1)

    pltpu.sync_copy(z_hbm, acc_sh.at[pl.ds(s * NROWS_S, NROWS_S)])

    @pl.when(c == 0)
    def _():
        pltpu.sync_copy(w_lo, w_v)

    @pl.when(c == 1)
    def _():
        pltpu.sync_copy(w_hi, w_v)

    plsc.subcore_barrier()

    def run(h_ref):
        w_regs = [w_v[pl.ds(16 * k, 16)] for k in range(DH // L)]

        def idx_start(blk, b):
            pltpu.make_async_copy(
                pk_hbm.at[s * BLKS + blk], pk[b], isem[b]).start()

        def idx_wait(b):
            pltpu.make_async_copy(pk_hbm.at[0], pk[b], isem[b]).wait()

        def gather_start(b):
            pltpu.make_async_copy(
                h_ref.at[pk[b].at[0]], rows[b], gsem[b]).start()

        def gather_wait(b):
            pltpu.make_async_copy(
                h_ref.at[pk[b].at[0]], rows[b], gsem[b]).wait()

        def scatter_start(b):
            for k in range(EBLK // L):
                db[b][0, pl.ds(k * L, L)] = pk[b][1, pl.ds(k * L, L)]
            pltpu.make_async_copy(
                rows[b], acc_sh.at[db[b].at[0]], ssem[b]).start(add=True)

        def scatter_wait(b):
            pltpu.make_async_copy(
                rows[b], acc_sh.at[db[b].at[0]], ssem[b]).wait()

        def compute(b):
            rows_v = rows[b]
            pk_v = pk[b]

            @plsc.parallel_loop(0, EBLK // L, unroll=2)
            def _(i16):
                len16 = plsc.bitcast(pk_v[2, pl.ds(i16 * L, L)], jnp.float32)
                for e in range(L):
                    bc = _bcast_lane(len16, e)
                    r = i16 * L + e
                    for seg in range(DH // L):
                        sl = rows_v[r, pl.ds(seg * L, L)]
                        rows_v[r, pl.ds(seg * L, L)] = jnp.maximum(
                            sl + bc * w_regs[seg], 0.0)

        idx_start(0, 0)
        idx_start(1, 1)
        idx_wait(0)
        gather_start(0)

        @pl.loop(0, BLKS, step=2)
        def _(blk):
            for b in range(2):
                i = blk + b
                bn = 1 - b
                gather_wait(b)

                @pl.when(i + 1 < BLKS)
                def _():
                    idx_wait(bn)

                    @pl.when(i >= 1)
                    def _():
                        scatter_wait(bn)

                    gather_start(bn)

                compute(b)
                scatter_start(b)

                @pl.when(i + 2 < BLKS)
                def _():
                    idx_start(i + 2, b)

        scatter_wait(0)
        scatter_wait(1)

    @pl.when(c == 0)
    def _():
        run(hb_lo)

    @pl.when(c == 1)
    def _():
        run(hb_hi)

    plsc.subcore_barrier()

    sl = pl.ds(s * NROWS_S, NROWS_S)

    @pl.when(c == 0)
    def _():
        pltpu.sync_copy(acc_sh.at[sl], out_lo.at[sl])

    @pl.when(c == 1)
    def _():
        pltpu.sync_copy(acc_sh.at[sl], out_hi.at[sl])


def _t1_body(a_ref, w_ref, b_ref, lo_ref, hi_ref):
    h = jnp.dot(a_ref[...], w_ref[...], preferred_element_type=jnp.float32)
    h = h + b_ref[...]
    lo_ref[...] = h[:, :128]
    hi_ref[...] = h[:, 128:]


def _t2_body(lo_ref, hi_ref, alo_ref, ahi_ref, gid_ref, wg_ref, bg_ref,
             be_ref, eps_ref, feat_ref):
    i = pl.program_id(0)

    @pl.when(i == 0)
    def _():
        feat_ref[...] = jnp.zeros_like(feat_ref)

    h = jnp.concatenate([lo_ref[...], hi_ref[...]], axis=1) - be_ref[...]
    a = jnp.concatenate([alo_ref[...], ahi_ref[...]], axis=1)
    x = (1.0 + eps_ref[0, 0]) * h + a
    t = jnp.dot(x, wg_ref[...], preferred_element_type=jnp.float32)
    gids = gid_ref[0, 0, :]
    rows = jax.lax.broadcasted_iota(jnp.int32, (B, ROWS), 0)
    mask = (gids[None, :] == rows).astype(jnp.float32)
    counts = jnp.sum(mask, axis=1)
    acc = jnp.dot(mask, t, preferred_element_type=jnp.float32)
    feat_ref[...] += acc + counts[:, None] * bg_ref[...]


def _t3_body(feat_ref, epsu_ref, wm_ref, bm_ref, out_ref):
    feat = feat_ref[...]
    mx = jnp.max(feat)
    mn = jnp.min(feat)
    f = (feat - mn) / (epsu_ref[0, 0] + mx - mn)
    out_ref[...] = jnp.dot(f, wm_ref[...],
                           preferred_element_type=jnp.float32) + bm_ref[...]


def kernel(atomic_num, edge_index, edge_length, graph_ids, W_node, b_node,
           W_edge, b_edge, W_gin, b_gin, eps_gin, eps_u, W_mlp, b_mlp):
    f32 = jnp.float32
    a_pad = jnp.zeros((NP_, KP), f32).at[:N, :ATOM].set(atomic_num)
    w_pad = jnp.zeros((KP, D), f32).at[:ATOM, :].set(W_node)

    hb_lo, hb_hi = pl.pallas_call(
        _t1_body,
        grid=(NBLK,),
        in_specs=[
            pl.BlockSpec((ROWS, KP), lambda i: (i, 0)),
            pl.BlockSpec((KP, D), lambda i: (0, 0)),
            pl.BlockSpec((1, D), lambda i: (0, 0)),
        ],
        out_specs=[
            pl.BlockSpec((ROWS, 128), lambda i: (i, 0)),
            pl.BlockSpec((ROWS, 128), lambda i: (i, 0)),
        ],
        out_shape=[
            jax.ShapeDtypeStruct((NP_, 128), f32),
            jax.ShapeDtypeStruct((NP_, 128), f32),
        ],
    )(a_pad, w_pad, (b_node + b_edge).reshape(1, D))

    src_pad = jnp.zeros((E_PAD,), jnp.int32).at[:E].set(edge_index[0])
    dst_pad = jnp.full((E_PAD,), N, jnp.int32).at[:E].set(edge_index[1])
    len_pad = jnp.zeros((E_PAD,), f32).at[:E].set(edge_length)
    packed = jnp.stack([
        src_pad.reshape(-1, EBLK),
        dst_pad.reshape(-1, EBLK),
        jax.lax.bitcast_convert_type(len_pad, jnp.int32).reshape(-1, EBLK),
    ], axis=1)
    zeros = jnp.zeros((NROWS_S, DH), f32)
    mesh = plsc.VectorSubcoreMesh(core_axis_name="c", subcore_axis_name="s")

    cp = pltpu.CompilerParams()
    if "needs_layout_passes" in pltpu.CompilerParams.__dataclass_fields__:
        cp = dataclasses.replace(cp, needs_layout_passes=False)

    edge_call = pl.kernel(
        _edge_body,
        compiler_params=cp,
        out_type=[
            jax.ShapeDtypeStruct((NP_, DH), f32),
            jax.ShapeDtypeStruct((NP_, DH), f32),
        ],
        mesh=mesh,
        scratch_types=[
            pltpu.VMEM((3, EBLK), jnp.int32),
            pltpu.VMEM((3, EBLK), jnp.int32),
            pltpu.VMEM((EBLK, DH), f32),
            pltpu.VMEM((EBLK, DH), f32),
            pltpu.VMEM((1, EBLK), jnp.int32),
            pltpu.VMEM((1, EBLK), jnp.int32),
            pltpu.VMEM((DH,), f32),
            pltpu.SemaphoreType.DMA,
            pltpu.SemaphoreType.DMA,
            pltpu.SemaphoreType.DMA,
            pltpu.SemaphoreType.DMA,
            pltpu.SemaphoreType.DMA,
            pltpu.SemaphoreType.DMA,
            pltpu.VMEM_SHARED((NP_, DH), f32),
        ],
    )
    agg_lo, agg_hi = edge_call(hb_lo, hb_hi, packed,
                               W_edge[0, :DH], W_edge[0, DH:], zeros)

    gids_pad = jnp.full((NP_,), B, jnp.int32).at[:N].set(graph_ids)
    gids3 = gids_pad.reshape(NBLK, 1, ROWS)

    feat = pl.pallas_call(
        _t2_body,
        grid=(NBLK,),
        in_specs=[
            pl.BlockSpec((ROWS, 128), lambda i: (i, 0)),
            pl.BlockSpec((ROWS, 128), lambda i: (i, 0)),
            pl.BlockSpec((ROWS, 128), lambda i: (i, 0)),
            pl.BlockSpec((ROWS, 128), lambda i: (i, 0)),
            pl.BlockSpec((1, 1, ROWS), lambda i: (i, 0, 0)),
            pl.BlockSpec((D, D), lambda i: (0, 0)),
            pl.BlockSpec((1, D), lambda i: (0, 0)),
            pl.BlockSpec((1, D), lambda i: (0, 0)),
            pl.BlockSpec((1, 1), lambda i: (0, 0)),
        ],
        out_specs=pl.BlockSpec((B, D), lambda i: (0, 0)),
        out_shape=jax.ShapeDtypeStruct((B, D), f32),
    )(hb_lo, hb_hi, agg_lo, agg_hi, gids3, W_gin, b_gin.reshape(1, D),
      b_edge.reshape(1, D), eps_gin.reshape(1, 1))

    wm_pad = jnp.zeros((D, 128), f32).at[:, :OUT].set(W_mlp)
    bm_pad = jnp.zeros((1, 128), f32).at[0, :OUT].set(b_mlp)

    out_pad = pl.pallas_call(
        _t3_body,
        in_specs=[
            pl.BlockSpec((B, D), lambda: (0, 0)),
            pl.BlockSpec((1, 1), lambda: (0, 0)),
            pl.BlockSpec((D, 128), lambda: (0, 0)),
            pl.BlockSpec((1, 128), lambda: (0, 0)),
        ],
        out_specs=pl.BlockSpec((B, 128), lambda: (0, 0)),
        out_shape=jax.ShapeDtypeStruct((B, 128), f32),
    )(feat, eps_u.reshape(1, 1), wm_pad, bm_pad)

    return out_pad[:, :OUT]


# speedup vs baseline: 1.6718x; 1.0554x over previous
import dataclasses
import functools

import jax
import jax.numpy as jnp
from jax import lax
from jax.experimental import pallas as pl
from jax.experimental.pallas import tpu as pltpu
from jax.experimental.pallas import tpu_sc as plsc

N = 10000
E = 160000
B = 64
ATOM = 118
D = 256
OUT = 100

NP_ = 10240
KP = 128
ROWS = 1024
NBLK = NP_ // ROWS

NC = 2
NS = 16
L = 16
DH = D // NC
EBLK = 64
BLKS = 160
E_PAD = NS * BLKS * EBLK
EPS_ = E_PAD // NS
NROWS_S = NP_ // NS


def _bcast_lane(vec16, e):
    idx = jnp.full((L, 1), e, jnp.int32)
    dn = lax.GatherDimensionNumbers(
        offset_dims=(), collapsed_slice_dims=(0,), start_index_map=(0,))
    return lax.gather(vec16, idx, dn, (1,),
                      mode=lax.GatherScatterMode.PROMISE_IN_BOUNDS)


def _edge_body(hb_lo, hb_hi, pk_hbm, w_lo, w_hi, z_hbm,
               out_lo, out_hi, pk0, pk1, pk2, pk3, rows0, rows1, rows2, rows3,
               db0, db1, db2, db3, w_v,
               isem0, isem1, isem2, isem3, gsem0, gsem1, gsem2, gsem3,
               ssem0, ssem1, ssem2, ssem3, acc_sh):
    c = lax.axis_index("c")
    s = lax.axis_index("s")
    pk = (pk0, pk1, pk2, pk3)
    db = (db0, db1, db2, db3)
    ssem = (ssem0, ssem1, ssem2, ssem3)
    rows = (rows0, rows1, rows2, rows3)
    isem = (isem0, isem1, isem2, isem3)
    gsem = (gsem0, gsem1, gsem2, gsem3)

    pltpu.sync_copy(z_hbm, acc_sh.at[pl.ds(s * NROWS_S, NROWS_S)])

    @pl.when(c == 0)
    def _():
        pltpu.sync_copy(w_lo, w_v)

    @pl.when(c == 1)
    def _():
        pltpu.sync_copy(w_hi, w_v)

    plsc.subcore_barrier()

    def run(h_ref):
        w_regs = [w_v[pl.ds(16 * k, 16)] for k in range(DH // L)]

        def idx_start(blk, b):
            pltpu.make_async_copy(
                pk_hbm.at[s * BLKS + blk], pk[b], isem[b]).start()

        def idx_wait(b):
            pltpu.make_async_copy(pk_hbm.at[0], pk[b], isem[b]).wait()

        def gather_start(b):
            pltpu.make_async_copy(
                h_ref.at[pk[b].at[0]], rows[b], gsem[b]).start()

        def gather_wait(b):
            pltpu.make_async_copy(
                h_ref.at[pk[b].at[0]], rows[b], gsem[b]).wait()

        def scatter_start(b):
            for k in range(EBLK // L):
                db[b][0, pl.ds(k * L, L)] = pk[b][1, pl.ds(k * L, L)]
            pltpu.make_async_copy(
                rows[b], acc_sh.at[db[b].at[0]], ssem[b]).start(add=True)

        def scatter_wait(b):
            pltpu.make_async_copy(
                rows[b], acc_sh.at[db[b].at[0]], ssem[b]).wait()

        def compute(b):
            rows_v = rows[b]
            pk_v = pk[b]

            @pl.loop(0, EBLK // L)
            def _(i16):
                len16 = plsc.bitcast(pk_v[2, pl.ds(i16 * L, L)], jnp.float32)

                @pl.loop(0, 4)
                def _(q):
                    for e4 in range(4):
                        e = q * 4 + e4
                        bc = _bcast_lane(len16, e)
                        r = i16 * L + e
                        for seg in range(DH // L):
                            sl = rows_v[r, pl.ds(seg * L, L)]
                            rows_v[r, pl.ds(seg * L, L)] = jnp.maximum(
                                sl + bc * w_regs[seg], 0.0)

        for b in range(4):
            idx_start(b, b)
        idx_wait(0)
        gather_start(0)
        idx_wait(1)
        gather_start(1)

        @pl.loop(0, BLKS, step=4)
        def _(blk):
            for b in range(4):
                i = blk + b
                bn = (b + 2) % 4
                gather_wait(b)

                @pl.when(i + 2 < BLKS)
                def _():
                    idx_wait(bn)

                    @pl.when(i >= 2)
                    def _():
                        scatter_wait(bn)

                    gather_start(bn)

                compute(b)
                scatter_start(b)

                @pl.when(i + 4 < BLKS)
                def _():
                    idx_start(i + 4, b)

        scatter_wait(2)
        scatter_wait(3)

    @pl.when(c == 0)
    def _():
        run(hb_lo)

    @pl.when(c == 1)
    def _():
        run(hb_hi)

    plsc.subcore_barrier()

    sl = pl.ds(s * NROWS_S, NROWS_S)

    @pl.when(c == 0)
    def _():
        pltpu.sync_copy(acc_sh.at[sl], out_lo.at[sl])

    @pl.when(c == 1)
    def _():
        pltpu.sync_copy(acc_sh.at[sl], out_hi.at[sl])


def _t1_body(a_ref, w_ref, b_ref, lo_ref, hi_ref):
    h = jnp.dot(a_ref[...], w_ref[...], preferred_element_type=jnp.float32)
    h = h + b_ref[...]
    lo_ref[...] = h[:, :128]
    hi_ref[...] = h[:, 128:]


def _t2_body(lo_ref, hi_ref, alo_ref, ahi_ref, gid_ref, wg_ref, bg_ref,
             be_ref, eps_ref, feat_ref):
    i = pl.program_id(0)

    @pl.when(i == 0)
    def _():
        feat_ref[...] = jnp.zeros_like(feat_ref)

    h = jnp.concatenate([lo_ref[...], hi_ref[...]], axis=1) - be_ref[...]
    a = jnp.concatenate([alo_ref[...], ahi_ref[...]], axis=1)
    x = (1.0 + eps_ref[0, 0]) * h + a
    t = jnp.dot(x, wg_ref[...], preferred_element_type=jnp.float32)
    gids = gid_ref[0, 0, :]
    rows = jax.lax.broadcasted_iota(jnp.int32, (B, ROWS), 0)
    mask = (gids[None, :] == rows).astype(jnp.float32)
    counts = jnp.sum(mask, axis=1)
    acc = jnp.dot(mask, t, preferred_element_type=jnp.float32)
    feat_ref[...] += acc + counts[:, None] * bg_ref[...]


def _t3_body(feat_ref, epsu_ref, wm_ref, bm_ref, out_ref):
    feat = feat_ref[...]
    mx = jnp.max(feat)
    mn = jnp.min(feat)
    f = (feat - mn) / (epsu_ref[0, 0] + mx - mn)
    out_ref[...] = jnp.dot(f, wm_ref[...],
                           preferred_element_type=jnp.float32) + bm_ref[...]


def kernel(atomic_num, edge_index, edge_length, graph_ids, W_node, b_node,
           W_edge, b_edge, W_gin, b_gin, eps_gin, eps_u, W_mlp, b_mlp):
    f32 = jnp.float32
    a_pad = jnp.zeros((NP_, KP), f32).at[:N, :ATOM].set(atomic_num)
    w_pad = jnp.zeros((KP, D), f32).at[:ATOM, :].set(W_node)

    hb_lo, hb_hi = pl.pallas_call(
        _t1_body,
        grid=(NBLK,),
        in_specs=[
            pl.BlockSpec((ROWS, KP), lambda i: (i, 0)),
            pl.BlockSpec((KP, D), lambda i: (0, 0)),
            pl.BlockSpec((1, D), lambda i: (0, 0)),
        ],
        out_specs=[
            pl.BlockSpec((ROWS, 128), lambda i: (i, 0)),
            pl.BlockSpec((ROWS, 128), lambda i: (i, 0)),
        ],
        out_shape=[
            jax.ShapeDtypeStruct((NP_, 128), f32),
            jax.ShapeDtypeStruct((NP_, 128), f32),
        ],
    )(a_pad, w_pad, (b_node + b_edge).reshape(1, D))

    src_pad = jnp.zeros((E_PAD,), jnp.int32).at[:E].set(edge_index[0])
    dst_pad = jnp.full((E_PAD,), N, jnp.int32).at[:E].set(edge_index[1])
    len_pad = jnp.zeros((E_PAD,), f32).at[:E].set(edge_length)
    packed = jnp.stack([
        src_pad.reshape(-1, EBLK),
        dst_pad.reshape(-1, EBLK),
        jax.lax.bitcast_convert_type(len_pad, jnp.int32).reshape(-1, EBLK),
    ], axis=1)
    zeros = jnp.zeros((NROWS_S, DH), f32)
    mesh = plsc.VectorSubcoreMesh(core_axis_name="c", subcore_axis_name="s")

    cp = pltpu.CompilerParams()
    if "needs_layout_passes" in pltpu.CompilerParams.__dataclass_fields__:
        cp = dataclasses.replace(cp, needs_layout_passes=False)

    edge_call = pl.kernel(
        _edge_body,
        compiler_params=cp,
        out_type=[
            jax.ShapeDtypeStruct((NP_, DH), f32),
            jax.ShapeDtypeStruct((NP_, DH), f32),
        ],
        mesh=mesh,
        scratch_types=(
            [pltpu.VMEM((3, EBLK), jnp.int32)] * 4
            + [pltpu.VMEM((EBLK, DH), f32)] * 4
            + [pltpu.VMEM((1, EBLK), jnp.int32)] * 4
            + [pltpu.VMEM((DH,), f32)]
            + [pltpu.SemaphoreType.DMA] * 12
            + [pltpu.VMEM_SHARED((NP_, DH), f32)]
        ),
    )
    agg_lo, agg_hi = edge_call(hb_lo, hb_hi, packed,
                               W_edge[0, :DH], W_edge[0, DH:], zeros)

    gids_pad = jnp.full((NP_,), B, jnp.int32).at[:N].set(graph_ids)
    gids3 = gids_pad.reshape(NBLK, 1, ROWS)

    feat = pl.pallas_call(
        _t2_body,
        grid=(NBLK,),
        in_specs=[
            pl.BlockSpec((ROWS, 128), lambda i: (i, 0)),
            pl.BlockSpec((ROWS, 128), lambda i: (i, 0)),
            pl.BlockSpec((ROWS, 128), lambda i: (i, 0)),
            pl.BlockSpec((ROWS, 128), lambda i: (i, 0)),
            pl.BlockSpec((1, 1, ROWS), lambda i: (i, 0, 0)),
            pl.BlockSpec((D, D), lambda i: (0, 0)),
            pl.BlockSpec((1, D), lambda i: (0, 0)),
            pl.BlockSpec((1, D), lambda i: (0, 0)),
            pl.BlockSpec((1, 1), lambda i: (0, 0)),
        ],
        out_specs=pl.BlockSpec((B, D), lambda i: (0, 0)),
        out_shape=jax.ShapeDtypeStruct((B, D), f32),
    )(hb_lo, hb_hi, agg_lo, agg_hi, gids3, W_gin, b_gin.reshape(1, D),
      b_edge.reshape(1, D), eps_gin.reshape(1, 1))

    wm_pad = jnp.zeros((D, 128), f32).at[:, :OUT].set(W_mlp)
    bm_pad = jnp.zeros((1, 128), f32).at[0, :OUT].set(b_mlp)

    out_pad = pl.pallas_call(
        _t3_body,
        in_specs=[
            pl.BlockSpec((B, D), lambda: (0, 0)),
            pl.BlockSpec((1, 1), lambda: (0, 0)),
            pl.BlockSpec((D, 128), lambda: (0, 0)),
            pl.BlockSpec((1, 128), lambda: (0, 0)),
        ],
        out_specs=pl.BlockSpec((B, 128), lambda: (0, 0)),
        out_shape=jax.ShapeDtypeStruct((B, 128), f32),
    )(feat, eps_u.reshape(1, 1), wm_pad, bm_pad)

    return out_pad[:, :OUT]


# speedup vs baseline: 1.6721x; 1.0002x over previous
import dataclasses
import functools

import jax
import jax.numpy as jnp
from jax import lax
from jax.experimental import pallas as pl
from jax.experimental.pallas import tpu as pltpu
from jax.experimental.pallas import tpu_sc as plsc

N = 10000
E = 160000
B = 64
ATOM = 118
D = 256
OUT = 100

NP_ = 10240
KP = 128
ROWS = 1024
NBLK = NP_ // ROWS

NC = 2
NS = 16
L = 16
DH = D // NC
EBLK = 64
BLKS = 160
E_PAD = NS * BLKS * EBLK
EPS_ = E_PAD // NS
NROWS_S = NP_ // NS


def _bcast_lane(vec16, e):
    idx = jnp.full((L, 1), e, jnp.int32)
    dn = lax.GatherDimensionNumbers(
        offset_dims=(), collapsed_slice_dims=(0,), start_index_map=(0,))
    return lax.gather(vec16, idx, dn, (1,),
                      mode=lax.GatherScatterMode.PROMISE_IN_BOUNDS)


def _edge_body(hb_lo, hb_hi, pk_hbm, w_lo, w_hi, z_hbm,
               out_lo, out_hi, pk0, pk1, pk2, pk3, rows0, rows1, rows2, rows3,
               db0, db1, db2, db3, w_v,
               isem0, isem1, isem2, isem3, gsem0, gsem1, gsem2, gsem3,
               ssem0, ssem1, ssem2, ssem3, acc_sh):
    c = lax.axis_index("c")
    s = lax.axis_index("s")
    pk = (pk0, pk1, pk2, pk3)
    db = (db0, db1, db2, db3)
    ssem = (ssem0, ssem1, ssem2, ssem3)
    rows = (rows0, rows1, rows2, rows3)
    isem = (isem0, isem1, isem2, isem3)
    gsem = (gsem0, gsem1, gsem2, gsem3)

    pltpu.sync_copy(z_hbm, acc_sh.at[pl.ds(s * NROWS_S, NROWS_S)])

    @pl.when(c == 0)
    def _():
        pltpu.sync_copy(w_lo, w_v)

    @pl.when(c == 1)
    def _():
        pltpu.sync_copy(w_hi, w_v)

    plsc.subcore_barrier()

    def run(h_ref):
        w_regs = [w_v[pl.ds(16 * k, 16)] for k in range(DH // L)]

        def idx_start(blk, b):
            pltpu.make_async_copy(
                pk_hbm.at[s * BLKS + blk], pk[b], isem[b]).start()

        def idx_wait(b):
            pltpu.make_async_copy(pk_hbm.at[0], pk[b], isem[b]).wait()

        def gather_start(b):
            pltpu.make_async_copy(
                h_ref.at[pk[b].at[0]], rows[b], gsem[b]).start()

        def gather_wait(b):
            pltpu.make_async_copy(
                h_ref.at[pk[b].at[0]], rows[b], gsem[b]).wait()

        def scatter_start(b):
            for k in range(EBLK // L):
                db[b][0, pl.ds(k * L, L)] = pk[b][1, pl.ds(k * L, L)]
            pltpu.make_async_copy(
                rows[b], acc_sh.at[db[b].at[0]], ssem[b]).start(add=True)

        def scatter_wait(b):
            pltpu.make_async_copy(
                rows[b], acc_sh.at[db[b].at[0]], ssem[b]).wait()

        def compute(b):
            rows_v = rows[b]
            pk_v = pk[b]

            @pl.loop(0, EBLK // L)
            def _(i16):
                len16 = plsc.bitcast(pk_v[2, pl.ds(i16 * L, L)], jnp.float32)

                @pl.loop(0, 4)
                def _(q):
                    for e4 in range(4):
                        e = q * 4 + e4
                        bc = _bcast_lane(len16, e)
                        r = i16 * L + e
                        for seg in range(DH // L):
                            sl = rows_v[r, pl.ds(seg * L, L)]
                            rows_v[r, pl.ds(seg * L, L)] = jnp.maximum(
                                sl + bc * w_regs[seg], 0.0)

        for b in range(4):
            idx_start(b, b)
        idx_wait(0)
        gather_start(0)
        idx_wait(1)
        gather_start(1)

        @pl.loop(0, BLKS, step=4)
        def _(blk):
            for b in range(4):
                i = blk + b
                bn = (b + 2) % 4
                gather_wait(b)

                @pl.when(i + 2 < BLKS)
                def _():
                    idx_wait(bn)

                    @pl.when(i >= 2)
                    def _():
                        scatter_wait(bn)

                    gather_start(bn)

                compute(b)
                scatter_start(b)

                @pl.when(i + 4 < BLKS)
                def _():
                    idx_start(i + 4, b)

        scatter_wait(0)
        scatter_wait(1)
        scatter_wait(2)
        scatter_wait(3)

    @pl.when(c == 0)
    def _():
        run(hb_lo)

    @pl.when(c == 1)
    def _():
        run(hb_hi)

    plsc.subcore_barrier()

    sl = pl.ds(s * NROWS_S, NROWS_S)

    @pl.when(c == 0)
    def _():
        pltpu.sync_copy(acc_sh.at[sl], out_lo.at[sl])

    @pl.when(c == 1)
    def _():
        pltpu.sync_copy(acc_sh.at[sl], out_hi.at[sl])


def _t1_body(a_ref, w_ref, b_ref, lo_ref, hi_ref):
    h = jnp.dot(a_ref[...], w_ref[...], preferred_element_type=jnp.float32)
    h = h + b_ref[...]
    lo_ref[...] = h[:, :128]
    hi_ref[...] = h[:, 128:]


def _t2_body(lo_ref, hi_ref, alo_ref, ahi_ref, gid_ref, wg_ref, bg_ref,
             be_ref, eps_ref, feat_ref):
    i = pl.program_id(0)

    @pl.when(i == 0)
    def _():
        feat_ref[...] = jnp.zeros_like(feat_ref)

    h = jnp.concatenate([lo_ref[...], hi_ref[...]], axis=1) - be_ref[...]
    a = jnp.concatenate([alo_ref[...], ahi_ref[...]], axis=1)
    x = (1.0 + eps_ref[0, 0]) * h + a
    t = jnp.dot(x, wg_ref[...], preferred_element_type=jnp.float32)
    gids = gid_ref[0, 0, :]
    rows = jax.lax.broadcasted_iota(jnp.int32, (B, ROWS), 0)
    mask = (gids[None, :] == rows).astype(jnp.float32)
    counts = jnp.sum(mask, axis=1)
    acc = jnp.dot(mask, t, preferred_element_type=jnp.float32)
    feat_ref[...] += acc + counts[:, None] * bg_ref[...]


def _t3_body(feat_ref, epsu_ref, wm_ref, bm_ref, out_ref):
    feat = feat_ref[...]
    mx = jnp.max(feat)
    mn = jnp.min(feat)
    f = (feat - mn) / (epsu_ref[0, 0] + mx - mn)
    out_ref[...] = jnp.dot(f, wm_ref[...],
                           preferred_element_type=jnp.float32) + bm_ref[...]


def kernel(atomic_num, edge_index, edge_length, graph_ids, W_node, b_node,
           W_edge, b_edge, W_gin, b_gin, eps_gin, eps_u, W_mlp, b_mlp):
    f32 = jnp.float32
    a_pad = jnp.zeros((NP_, KP), f32).at[:N, :ATOM].set(atomic_num)
    w_pad = jnp.zeros((KP, D), f32).at[:ATOM, :].set(W_node)

    hb_lo, hb_hi = pl.pallas_call(
        _t1_body,
        grid=(NBLK,),
        in_specs=[
            pl.BlockSpec((ROWS, KP), lambda i: (i, 0)),
            pl.BlockSpec((KP, D), lambda i: (0, 0)),
            pl.BlockSpec((1, D), lambda i: (0, 0)),
        ],
        out_specs=[
            pl.BlockSpec((ROWS, 128), lambda i: (i, 0)),
            pl.BlockSpec((ROWS, 128), lambda i: (i, 0)),
        ],
        out_shape=[
            jax.ShapeDtypeStruct((NP_, 128), f32),
            jax.ShapeDtypeStruct((NP_, 128), f32),
        ],
    )(a_pad, w_pad, (b_node + b_edge).reshape(1, D))

    src_pad = jnp.zeros((E_PAD,), jnp.int32).at[:E].set(edge_index[0])
    dst_pad = jnp.full((E_PAD,), N, jnp.int32).at[:E].set(edge_index[1])
    len_pad = jnp.zeros((E_PAD,), f32).at[:E].set(edge_length)
    packed = jnp.stack([
        src_pad.reshape(-1, EBLK),
        dst_pad.reshape(-1, EBLK),
        jax.lax.bitcast_convert_type(len_pad, jnp.int32).reshape(-1, EBLK),
    ], axis=1)
    zeros = jnp.zeros((NROWS_S, DH), f32)
    mesh = plsc.VectorSubcoreMesh(core_axis_name="c", subcore_axis_name="s")

    cp = pltpu.CompilerParams()
    if "needs_layout_passes" in pltpu.CompilerParams.__dataclass_fields__:
        cp = dataclasses.replace(cp, needs_layout_passes=False)

    edge_call = pl.kernel(
        _edge_body,
        compiler_params=cp,
        out_type=[
            jax.ShapeDtypeStruct((NP_, DH), f32),
            jax.ShapeDtypeStruct((NP_, DH), f32),
        ],
        mesh=mesh,
        scratch_types=(
            [pltpu.VMEM((3, EBLK), jnp.int32)] * 4
            + [pltpu.VMEM((EBLK, DH), f32)] * 4
            + [pltpu.VMEM((1, EBLK), jnp.int32)] * 4
            + [pltpu.VMEM((DH,), f32)]
            + [pltpu.SemaphoreType.DMA] * 12
            + [pltpu.VMEM_SHARED((NP_, DH), f32)]
        ),
    )
    agg_lo, agg_hi = edge_call(hb_lo, hb_hi, packed,
                               W_edge[0, :DH], W_edge[0, DH:], zeros)

    gids_pad = jnp.full((NP_,), B, jnp.int32).at[:N].set(graph_ids)
    gids3 = gids_pad.reshape(NBLK, 1, ROWS)

    feat = pl.pallas_call(
        _t2_body,
        grid=(NBLK,),
        in_specs=[
            pl.BlockSpec((ROWS, 128), lambda i: (i, 0)),
            pl.BlockSpec((ROWS, 128), lambda i: (i, 0)),
            pl.BlockSpec((ROWS, 128), lambda i: (i, 0)),
            pl.BlockSpec((ROWS, 128), lambda i: (i, 0)),
            pl.BlockSpec((1, 1, ROWS), lambda i: (i, 0, 0)),
            pl.BlockSpec((D, D), lambda i: (0, 0)),
            pl.BlockSpec((1, D), lambda i: (0, 0)),
            pl.BlockSpec((1, D), lambda i: (0, 0)),
            pl.BlockSpec((1, 1), lambda i: (0, 0)),
        ],
        out_specs=pl.BlockSpec((B, D), lambda i: (0, 0)),
        out_shape=jax.ShapeDtypeStruct((B, D), f32),
    )(hb_lo, hb_hi, agg_lo, agg_hi, gids3, W_gin, b_gin.reshape(1, D),
      b_edge.reshape(1, D), eps_gin.reshape(1, 1))

    wm_pad = jnp.zeros((D, 128), f32).at[:, :OUT].set(W_mlp)
    bm_pad = jnp.zeros((1, 128), f32).at[0, :OUT].set(b_mlp)

    out_pad = pl.pallas_call(
        _t3_body,
        in_specs=[
            pl.BlockSpec((B, D), lambda: (0, 0)),
            pl.BlockSpec((1, 1), lambda: (0, 0)),
            pl.BlockSpec((D, 128), lambda: (0, 0)),
            pl.BlockSpec((1, 128), lambda: (0, 0)),
        ],
        out_specs=pl.BlockSpec((B, 128), lambda: (0, 0)),
        out_shape=jax.ShapeDtypeStruct((B, 128), f32),
    )(feat, eps_u.reshape(1, 1), wm_pad, bm_pad)

    return out_pad[:, :OUT]


# speedup vs baseline: 1.6783x; 1.0037x over previous
import dataclasses
import functools

import jax
import jax.numpy as jnp
from jax import lax
from jax.experimental import pallas as pl
from jax.experimental.pallas import tpu as pltpu
from jax.experimental.pallas import tpu_sc as plsc

N = 10000
E = 160000
B = 64
ATOM = 118
D = 256
OUT = 100

NP_ = 10240
KP = 128
ROWS = 1024
NBLK = NP_ // ROWS

NC = 2
NS = 16
L = 16
DH = D // NC
EBLK = 64
BLKS = 160
E_PAD = NS * BLKS * EBLK
EPS_ = E_PAD // NS
NROWS_S = NP_ // NS


def _bcast_lane(vec16, e):
    idx = jnp.full((L, 1), e, jnp.int32)
    dn = lax.GatherDimensionNumbers(
        offset_dims=(), collapsed_slice_dims=(0,), start_index_map=(0,))
    return lax.gather(vec16, idx, dn, (1,),
                      mode=lax.GatherScatterMode.PROMISE_IN_BOUNDS)


def _edge_body(hb_lo, hb_hi, pk_hbm, w_lo, w_hi, z_hbm,
               out_lo, out_hi, pk0, pk1, pk2, pk3, rows0, rows1, rows2, rows3,
               db0, db1, db2, db3, w_v,
               isem0, isem1, isem2, isem3, gsem0, gsem1, gsem2, gsem3,
               ssem0, ssem1, ssem2, ssem3, acc_sh):
    c = lax.axis_index("c")
    s = lax.axis_index("s")
    pk = (pk0, pk1, pk2, pk3)
    db = (db0, db1, db2, db3)
    ssem = (ssem0, ssem1, ssem2, ssem3)
    rows = (rows0, rows1, rows2, rows3)
    isem = (isem0, isem1, isem2, isem3)
    gsem = (gsem0, gsem1, gsem2, gsem3)

    pltpu.sync_copy(z_hbm, acc_sh.at[pl.ds(s * NROWS_S, NROWS_S)])

    @pl.when(c == 0)
    def _():
        pltpu.sync_copy(w_lo, w_v)

    @pl.when(c == 1)
    def _():
        pltpu.sync_copy(w_hi, w_v)

    plsc.subcore_barrier()

    def run(h_ref):
        w_regs = [w_v[pl.ds(16 * k, 16)] for k in range(DH // L)]

        def idx_start(blk, b):
            pltpu.make_async_copy(
                pk_hbm.at[s * BLKS + blk], pk[b], isem[b]).start()

        def idx_wait(b):
            pltpu.make_async_copy(pk_hbm.at[0], pk[b], isem[b]).wait()

        def gather_start(b):
            pltpu.make_async_copy(
                h_ref.at[pk[b].at[0]], rows[b], gsem[b]).start()

        def gather_wait(b):
            pltpu.make_async_copy(
                h_ref.at[pk[b].at[0]], rows[b], gsem[b]).wait()

        def scatter_start(b):
            for k in range(EBLK // L):
                db[b][0, pl.ds(k * L, L)] = pk[b][1, pl.ds(k * L, L)]
            pltpu.make_async_copy(
                rows[b], acc_sh.at[db[b].at[0]], ssem[b]).start(add=True)

        def scatter_wait(b):
            pltpu.make_async_copy(
                rows[b], acc_sh.at[db[b].at[0]], ssem[b]).wait()

        def compute(b):
            rows_v = rows[b]
            pk_v = pk[b]

            @pl.loop(0, EBLK // L)
            def _(i16):
                len16 = plsc.bitcast(pk_v[2, pl.ds(i16 * L, L)], jnp.float32)

                @pl.loop(0, 4)
                def _(q):
                    for e4 in range(4):
                        e = q * 4 + e4
                        bc = _bcast_lane(len16, e)
                        r = i16 * L + e
                        for seg in range(DH // L):
                            sl = rows_v[r, pl.ds(seg * L, L)]
                            rows_v[r, pl.ds(seg * L, L)] = jnp.maximum(
                                sl + bc * w_regs[seg], 0.0)

        for b in range(4):
            idx_start(b, b)
        idx_wait(0)
        gather_start(0)
        idx_wait(1)
        gather_start(1)

        @pl.loop(0, BLKS, step=4)
        def _(blk):
            for b in range(4):
                i = blk + b
                bn = (b + 2) % 4
                gather_wait(b)

                @pl.when(i + 2 < BLKS)
                def _():
                    idx_wait(bn)

                    @pl.when(i >= 2)
                    def _():
                        scatter_wait(bn)

                    gather_start(bn)

                compute(b)
                scatter_start(b)

                @pl.when(i + 4 < BLKS)
                def _():
                    idx_start(i + 4, b)

        scatter_wait(0)
        scatter_wait(1)
        scatter_wait(2)
        scatter_wait(3)

    @pl.when(c == 0)
    def _():
        run(hb_lo)

    @pl.when(c == 1)
    def _():
        run(hb_hi)

    plsc.subcore_barrier()

    sl = pl.ds(s * NROWS_S, NROWS_S)

    @pl.when(c == 0)
    def _():
        pltpu.sync_copy(acc_sh.at[sl], out_lo.at[sl])

    @pl.when(c == 1)
    def _():
        pltpu.sync_copy(acc_sh.at[sl], out_hi.at[sl])


def _t1_body(a_ref, w_ref, b_ref, lo_ref, hi_ref):
    h = jnp.dot(a_ref[...], w_ref[...], preferred_element_type=jnp.float32)
    h = h + b_ref[...]
    lo_ref[...] = h[:, :128]
    hi_ref[...] = h[:, 128:]


def _t2_body(lo_ref, hi_ref, alo_ref, ahi_ref, gid_ref, wg_ref, bg_ref,
             be_ref, eps_ref, epsu_ref, wm_ref, bm_ref, out_ref, feat_ref):
    i = pl.program_id(0)

    @pl.when(i == 0)
    def _():
        feat_ref[...] = jnp.zeros_like(feat_ref)

    h = jnp.concatenate([lo_ref[...], hi_ref[...]], axis=1) - be_ref[...]
    a = jnp.concatenate([alo_ref[...], ahi_ref[...]], axis=1)
    x = (1.0 + eps_ref[0, 0]) * h + a
    t = jnp.dot(x, wg_ref[...], preferred_element_type=jnp.float32)
    gids = gid_ref[0, 0, :]
    rows = jax.lax.broadcasted_iota(jnp.int32, (B, ROWS), 0)
    mask = (gids[None, :] == rows).astype(jnp.float32)
    counts = jnp.sum(mask, axis=1)
    acc = jnp.dot(mask, t, preferred_element_type=jnp.float32)
    feat_ref[...] += acc + counts[:, None] * bg_ref[...]

    @pl.when(i == NBLK - 1)
    def _():
        feat = feat_ref[...]
        mx = jnp.max(feat)
        mn = jnp.min(feat)
        f = (feat - mn) / (epsu_ref[0, 0] + mx - mn)
        out_ref[...] = jnp.dot(
            f, wm_ref[...], preferred_element_type=jnp.float32) + bm_ref[...]


def kernel(atomic_num, edge_index, edge_length, graph_ids, W_node, b_node,
           W_edge, b_edge, W_gin, b_gin, eps_gin, eps_u, W_mlp, b_mlp):
    f32 = jnp.float32
    a_pad = jnp.zeros((NP_, KP), f32).at[:N, :ATOM].set(atomic_num)
    w_pad = jnp.zeros((KP, D), f32).at[:ATOM, :].set(W_node)

    hb_lo, hb_hi = pl.pallas_call(
        _t1_body,
        grid=(NBLK,),
        in_specs=[
            pl.BlockSpec((ROWS, KP), lambda i: (i, 0)),
            pl.BlockSpec((KP, D), lambda i: (0, 0)),
            pl.BlockSpec((1, D), lambda i: (0, 0)),
        ],
        out_specs=[
            pl.BlockSpec((ROWS, 128), lambda i: (i, 0)),
            pl.BlockSpec((ROWS, 128), lambda i: (i, 0)),
        ],
        out_shape=[
            jax.ShapeDtypeStruct((NP_, 128), f32),
            jax.ShapeDtypeStruct((NP_, 128), f32),
        ],
    )(a_pad, w_pad, (b_node + b_edge).reshape(1, D))

    src_pad = jnp.zeros((E_PAD,), jnp.int32).at[:E].set(edge_index[0])
    dst_pad = jnp.full((E_PAD,), N, jnp.int32).at[:E].set(edge_index[1])
    len_pad = jnp.zeros((E_PAD,), f32).at[:E].set(edge_length)
    packed = jnp.stack([
        src_pad.reshape(-1, EBLK),
        dst_pad.reshape(-1, EBLK),
        jax.lax.bitcast_convert_type(len_pad, jnp.int32).reshape(-1, EBLK),
    ], axis=1)
    zeros = jnp.zeros((NROWS_S, DH), f32)
    mesh = plsc.VectorSubcoreMesh(core_axis_name="c", subcore_axis_name="s")

    cp = pltpu.CompilerParams()
    if "needs_layout_passes" in pltpu.CompilerParams.__dataclass_fields__:
        cp = dataclasses.replace(cp, needs_layout_passes=False)

    edge_call = pl.kernel(
        _edge_body,
        compiler_params=cp,
        out_type=[
            jax.ShapeDtypeStruct((NP_, DH), f32),
            jax.ShapeDtypeStruct((NP_, DH), f32),
        ],
        mesh=mesh,
        scratch_types=(
            [pltpu.VMEM((3, EBLK), jnp.int32)] * 4
            + [pltpu.VMEM((EBLK, DH), f32)] * 4
            + [pltpu.VMEM((1, EBLK), jnp.int32)] * 4
            + [pltpu.VMEM((DH,), f32)]
            + [pltpu.SemaphoreType.DMA] * 12
            + [pltpu.VMEM_SHARED((NP_, DH), f32)]
        ),
    )
    agg_lo, agg_hi = edge_call(hb_lo, hb_hi, packed,
                               W_edge[0, :DH], W_edge[0, DH:], zeros)

    gids_pad = jnp.full((NP_,), B, jnp.int32).at[:N].set(graph_ids)
    gids3 = gids_pad.reshape(NBLK, 1, ROWS)

    wm_pad = jnp.zeros((D, 128), f32).at[:, :OUT].set(W_mlp)
    bm_pad = jnp.zeros((1, 128), f32).at[0, :OUT].set(b_mlp)

    out_pad = pl.pallas_call(
        _t2_body,
        grid=(NBLK,),
        in_specs=[
            pl.BlockSpec((ROWS, 128), lambda i: (i, 0)),
            pl.BlockSpec((ROWS, 128), lambda i: (i, 0)),
            pl.BlockSpec((ROWS, 128), lambda i: (i, 0)),
            pl.BlockSpec((ROWS, 128), lambda i: (i, 0)),
            pl.BlockSpec((1, 1, ROWS), lambda i: (i, 0, 0)),
            pl.BlockSpec((D, D), lambda i: (0, 0)),
            pl.BlockSpec((1, D), lambda i: (0, 0)),
            pl.BlockSpec((1, D), lambda i: (0, 0)),
            pl.BlockSpec((1, 1), lambda i: (0, 0)),
            pl.BlockSpec((1, 1), lambda i: (0, 0)),
            pl.BlockSpec((D, 128), lambda i: (0, 0)),
            pl.BlockSpec((1, 128), lambda i: (0, 0)),
        ],
        out_specs=pl.BlockSpec((B, 128), lambda i: (0, 0)),
        out_shape=jax.ShapeDtypeStruct((B, 128), f32),
        scratch_shapes=[pltpu.VMEM((B, D), jnp.float32)],
    )(hb_lo, hb_hi, agg_lo, agg_hi, gids3, W_gin, b_gin.reshape(1, D),
      b_edge.reshape(1, D), eps_gin.reshape(1, 1), eps_u.reshape(1, 1),
      wm_pad, bm_pad)

    return out_pad[:, :OUT]
